# Initial kernel scaffold; baseline (speedup 1.0000x reference)
import functools

import jax
import jax.numpy as jnp
from jax import lax
from jax.experimental import pallas as pl
from jax.experimental.pallas import tpu as pltpu
from jax.experimental.pallas import tpu_sc as plsc

NC = 2
NS = 16
NW = NC * NS
CHUNK = 128

N_ELEM = 4
NB = 8
RC = 5.0
F = 32
FV = 8
INV_SQRT_DEG = 0.25


def _sc_gather(tab, src, dst):
    ep = src.shape[0]
    per_w = ep // NW
    n_chunks = per_w // CHUNK
    mesh = plsc.VectorSubcoreMesh(
        core_axis_name="c", subcore_axis_name="s", num_cores=NC,
        num_subcores=NS)

    def body(tab_hbm, src_hbm, dst_hbm, osrc_hbm, odst_hbm,
             idx_v, rows_v, sem):
        wid = lax.axis_index("s") * NC + lax.axis_index("c")
        base_w = wid * per_w

        def chunk_body(j, carry):
            base = base_w + j * CHUNK
            pltpu.sync_copy(src_hbm.at[pl.ds(base, CHUNK)], idx_v)
            pltpu.async_copy(tab_hbm.at[idx_v], rows_v, sem).wait()
            pltpu.sync_copy(rows_v, osrc_hbm.at[pl.ds(base, CHUNK)])
            pltpu.sync_copy(dst_hbm.at[pl.ds(base, CHUNK)], idx_v)
            pltpu.async_copy(tab_hbm.at[idx_v], rows_v, sem).wait()
            pltpu.sync_copy(rows_v, odst_hbm.at[pl.ds(base, CHUNK)])
            return carry

        lax.fori_loop(0, n_chunks, chunk_body, 0)

    f = pl.kernel(
        body,
        out_type=[jax.ShapeDtypeStruct((ep, 8), jnp.float32),
                  jax.ShapeDtypeStruct((ep, 8), jnp.float32)],
        mesh=mesh,
        scratch_types=[pltpu.VMEM((CHUNK,), jnp.int32),
                       pltpu.VMEM((CHUNK, 8), jnp.float32),
                       pltpu.SemaphoreType.DMA],
    )
    return f(tab, src, dst)


def _sc_scatter(msg_s, msg_v, dst, zrows, n_nodes):
    ep = dst.shape[0]
    per_t = ep // NS
    n_chunks = per_t // CHUNK
    rows_per_t = n_nodes // NS
    mesh = plsc.VectorSubcoreMesh(
        core_axis_name="c", subcore_axis_name="s", num_cores=NC,
        num_subcores=NS)

    def body(ms_hbm, mv_hbm, dst_hbm, z_hbm, outs_hbm, outv_hbm,
             acc, idx_v, buf, zbuf, sem):
        c = lax.axis_index("c")
        s = lax.axis_index("s")
        pltpu.sync_copy(z_hbm, zbuf)
        pltpu.sync_copy(zbuf, acc.at[pl.ds(s * rows_per_t, rows_per_t)])
        plsc.subcore_barrier()

        base_t = s * per_t

        def chunk_body(j, carry):
            base = base_t + j * CHUNK
            pltpu.sync_copy(dst_hbm.at[pl.ds(base, CHUNK)], idx_v)

            @pl.when(c == 0)
            def _():
                pltpu.sync_copy(ms_hbm.at[pl.ds(base, CHUNK)], buf)

            @pl.when(c == 1)
            def _():
                pltpu.sync_copy(mv_hbm.at[pl.ds(base, CHUNK)], buf)

            pltpu.sync_copy(buf, acc.at[idx_v], add=True)
            return carry

        lax.fori_loop(0, n_chunks, chunk_body, 0)
        plsc.subcore_barrier()

        pltpu.sync_copy(acc.at[pl.ds(s * rows_per_t, rows_per_t)], zbuf)

        @pl.when(c == 0)
        def _():
            pltpu.sync_copy(zbuf, outs_hbm.at[pl.ds(s * rows_per_t,
                                                    rows_per_t)])

        @pl.when(c == 1)
        def _():
            pltpu.sync_copy(zbuf, outv_hbm.at[pl.ds(s * rows_per_t,
                                                    rows_per_t)])

    f = pl.kernel(
        body,
        out_type=[jax.ShapeDtypeStruct((n_nodes, 32), jnp.float32),
                  jax.ShapeDtypeStruct((n_nodes, 32), jnp.float32)],
        mesh=mesh,
        scratch_types=[pltpu.VMEM_SHARED((n_nodes, 32), jnp.float32),
                       pltpu.VMEM((CHUNK,), jnp.int32),
                       pltpu.VMEM((CHUNK, 32), jnp.float32),
                       pltpu.VMEM((rows_per_t, 32), jnp.float32),
                       pltpu.SemaphoreType.DMA],
    )
    return f(msg_s, msg_v, dst, zrows)


def _edge_body(e_real, be, src_ref, dst_ref, wemb_ref, w0_ref, w1_ref,
               w2_ref, ms_ref, mv_ref):
    i = pl.program_id(0)
    a = src_ref[...]
    b = dst_ref[...]
    d = b[:, 0:3] - a[:, 0:3]
    r = jnp.sqrt(jnp.sum(d * d, axis=1, keepdims=True) + 1e-12)
    unit = d / (r + 1e-8)
    sh1 = jnp.sqrt(3.0) * unit
    rs = r + 1e-8
    n = lax.broadcasted_iota(jnp.float32, (1, NB), 1) + 1.0
    bes = jnp.sqrt(2.0 / RC) * jnp.sin(n * (jnp.pi / RC) * rs) / rs
    u = jnp.clip(r / RC, 0.0, 1.0)
    env = 1.0 - 28.0 * u**6 + 48.0 * u**7 - 21.0 * u**8
    env = jnp.where(u < 1.0, env, 0.0)
    ea = bes * env
    h = jnp.tanh(jnp.dot(ea, w0_ref[...], preferred_element_type=jnp.float32))
    h = jnp.tanh(jnp.dot(h, w1_ref[...], preferred_element_type=jnp.float32))
    w = jnp.dot(h, w2_ref[...], preferred_element_type=jnp.float32)
    spec = a[:, 3:4]
    oh = (spec == lax.broadcasted_iota(jnp.float32, (1, N_ELEM), 1))
    s_src = jnp.dot(oh.astype(jnp.float32), wemb_ref[...],
                    preferred_element_type=jnp.float32)
    ids = i * be + lax.broadcasted_iota(jnp.int32, (be, 1), 0)
    valid = (ids < e_real).astype(jnp.float32)
    ms_ref[...] = w[:, 0:F] * s_src * INV_SQRT_DEG * valid
    mv8 = w[:, F:F + FV] * s_src[:, 0:FV] * INV_SQRT_DEG
    zero8 = jnp.zeros((be, FV), jnp.float32)
    mv = jnp.concatenate(
        [mv8 * sh1[:, 0:1], mv8 * sh1[:, 1:2], mv8 * sh1[:, 2:3], zero8],
        axis=1)
    mv_ref[...] = mv * valid


def _tc_edge(srcrow, dstrow, w_emb, w_r0, w_r1, w_r2, e_real):
    ep = srcrow.shape[0]
    be = 4096
    grid = ep // be
    full = lambda shape: pl.BlockSpec(shape, lambda i: (0, 0))
    return pl.pallas_call(
        functools.partial(_edge_body, e_real, be),
        grid=(grid,),
        in_specs=[pl.BlockSpec((be, 8), lambda i: (i, 0)),
                  pl.BlockSpec((be, 8), lambda i: (i, 0)),
                  full(w_emb.shape), full(w_r0.shape), full(w_r1.shape),
                  full(w_r2.shape)],
        out_specs=[pl.BlockSpec((be, 32), lambda i: (i, 0)),
                   pl.BlockSpec((be, 32), lambda i: (i, 0))],
        out_shape=[jax.ShapeDtypeStruct((ep, 32), jnp.float32),
                   jax.ShapeDtypeStruct((ep, 32), jnp.float32)],
    )(srcrow, dstrow, w_emb, w_r0, w_r1, w_r2)


def _norm_act(x):
    n = jnp.abs(x)
    return x * jnp.tanh(n) / (n + 1e-8)


def _node_body(sp_ref, as_ref, av_ref, wattr, wemb, wsc, wouts, wv2s, wfin,
               wrnl1, wm0, wm1, wm2, out_ref):
    sp = sp_ref[...]
    oh = (sp == lax.broadcasted_iota(jnp.int32, (1, N_ELEM), 1)).astype(
        jnp.float32)
    dot = lambda x, w: jnp.dot(x, w[...], preferred_element_type=jnp.float32)
    s0 = dot(oh, wemb)
    na = dot(oh, wattr)
    sc = dot(s0 * na, wsc)
    av = av_ref[...]
    vnorm = jnp.sqrt(av[:, 0:8]**2 + av[:, 8:16]**2 + av[:, 16:24]**2 + 1e-8)
    s_out = sc + dot(as_ref[...], wouts) + dot(vnorm, wv2s)
    s_out = _norm_act(s_out)
    x = _norm_act(dot(s_out, wfin))
    x = x + _norm_act(dot(x, wrnl1))
    hm = jnp.tanh(dot(x, wm0) * (1.0 / jnp.sqrt(32.0)))
    hm = jnp.tanh(dot(hm, wm1) * (1.0 / jnp.sqrt(64.0)))
    out_ref[...] = dot(hm, wm2) * (1.0 / jnp.sqrt(32.0))


def _tc_node(species2d, agg_s, agg_v, w_attr, w_emb, w_sc, w_out_s, w_v2s,
             w_fin, w_rnl1, w_m0, w_m1, w_m2):
    n = species2d.shape[0]
    bn = 2000
    grid = n // bn
    full = lambda arr: pl.BlockSpec(arr.shape, lambda i: (0, 0))
    return pl.pallas_call(
        _node_body,
        grid=(grid,),
        in_specs=[pl.BlockSpec((bn, 1), lambda i: (i, 0)),
                  pl.BlockSpec((bn, 32), lambda i: (i, 0)),
                  pl.BlockSpec((bn, 32), lambda i: (i, 0)),
                  full(w_attr), full(w_emb), full(w_sc), full(w_out_s),
                  full(w_v2s), full(w_fin), full(w_rnl1), full(w_m0),
                  full(w_m1), full(w_m2)],
        out_specs=pl.BlockSpec((bn, 1), lambda i: (i, 0)),
        out_shape=jax.ShapeDtypeStruct((n, 1), jnp.float32),
    )(species2d, agg_s, agg_v, w_attr, w_emb, w_sc, w_out_s, w_v2s, w_fin,
      w_rnl1, w_m0, w_m1, w_m2)


def kernel(species, coords, edge_index, contributions, W_attr, W_emb, W_r0,
           W_r1, W_r2, W_sc, W_out_s, W_v2s, W_fin, W_rnl1, W_m0, W_m1,
           W_m2):
    n = coords.shape[0]
    e = edge_index.shape[1]
    tab = jnp.concatenate(
        [coords.astype(jnp.float32),
         species.astype(jnp.float32)[:, None],
         jnp.zeros((n, 4), jnp.float32)], axis=1)
    src = edge_index[0].astype(jnp.int32)
    dst = edge_index[1].astype(jnp.int32)
    align = NW * CHUNK
    ep = ((e + align - 1) // align) * align
    if ep != e:
        src = jnp.concatenate([src, jnp.zeros((ep - e,), jnp.int32)])
        dst = jnp.concatenate([dst, jnp.zeros((ep - e,), jnp.int32)])
    srcrow, dstrow = _sc_gather(tab, src, dst)
    ms, mv = _tc_edge(srcrow, dstrow, W_emb, W_r0, W_r1, W_r2, e)
    zrows = jnp.zeros((n // NS, 32), jnp.float32)
    agg_s, agg_v = _sc_scatter(ms, mv, dst, zrows, n)
    out = _tc_node(species[:, None].astype(jnp.int32), agg_s, agg_v, W_attr,
                   W_emb, W_sc, W_out_s, W_v2s, W_fin, W_rnl1, W_m0, W_m1,
                   W_m2)
    return out


# speedup vs baseline: 19.0615x; 19.0615x over previous
import functools

import jax
import jax.numpy as jnp
from jax import lax
from jax.experimental import pallas as pl
from jax.experimental.pallas import tpu as pltpu
from jax.experimental.pallas import tpu_sc as plsc

NC = 2
NS = 16
NW = NC * NS
CHUNK = 128

N_ELEM = 4
NB = 8
RC = 5.0
F = 32
FV = 8
INV_SQRT_DEG = 0.25


def _sc_gather(tab, edge_index):
    ep = edge_index.shape[0] // 2
    per_w = ep // NW
    n_full = per_w // CHUNK
    has_tail = (per_w % CHUNK) != 0
    mesh = plsc.VectorSubcoreMesh(
        core_axis_name="c", subcore_axis_name="s", num_cores=NC,
        num_subcores=NS)

    def body(tab_hbm, ei_hbm, osrc_hbm, odst_hbm, idx_v, rows_v, sem):
        wid = lax.axis_index("s") * NC + lax.axis_index("c")
        base_w = wid * per_w

        def do_chunk(base):
            pltpu.sync_copy(ei_hbm.at[pl.ds(base, CHUNK)], idx_v)
            pltpu.async_copy(tab_hbm.at[idx_v], rows_v, sem).wait()
            pltpu.sync_copy(rows_v, osrc_hbm.at[pl.ds(base, CHUNK)])
            pltpu.sync_copy(ei_hbm.at[pl.ds(ep + base, CHUNK)], idx_v)
            pltpu.async_copy(tab_hbm.at[idx_v], rows_v, sem).wait()
            pltpu.sync_copy(rows_v, odst_hbm.at[pl.ds(base, CHUNK)])

        def chunk_body(j, carry):
            do_chunk(base_w + j * CHUNK)
            return carry

        lax.fori_loop(0, n_full, chunk_body, 0)
        if has_tail:
            do_chunk(base_w + per_w - CHUNK)

    f = pl.kernel(
        body,
        out_type=[pltpu.HBM((ep, 8), jnp.float32),
                  pltpu.HBM((ep, 8), jnp.float32)],
        mesh=mesh,
        scratch_types=[pltpu.VMEM((CHUNK,), jnp.int32),
                       pltpu.VMEM((CHUNK, 8), jnp.float32),
                       pltpu.SemaphoreType.DMA],
        compiler_params=pltpu.CompilerParams(use_tc_tiling_on_sc=False),
    )
    return f(tab, edge_index)


def _sc_scatter(msg_s, msg_v, dst, zrows, n_nodes):
    ep = dst.shape[0] // 2
    per_t = ep // NS
    n_full = per_t // CHUNK
    tail = per_t % CHUNK
    rows_t = n_nodes // NS
    wrow = 125
    n_wchunk = rows_t // wrow
    mesh = plsc.VectorSubcoreMesh(
        core_axis_name="c", subcore_axis_name="s", num_cores=NC,
        num_subcores=NS)

    def body(ms_hbm, mv_hbm, dst_hbm, z_hbm, outs_hbm, outv_hbm,
             acc, idx_v, buf, idx_t, buf_t, wbuf, sem):
        c = lax.axis_index("c")
        s = lax.axis_index("s")
        base_t = s * per_t
        row0 = s * rows_t

        pltpu.sync_copy(z_hbm, wbuf)

        def zero_body(q, carry):
            pltpu.sync_copy(wbuf, acc.at[pl.ds(row0 + q * wrow, wrow)])
            return carry

        lax.fori_loop(0, n_wchunk, zero_body, 0)
        plsc.subcore_barrier()

        def do_chunk(base, size, idx, dat):
            pltpu.sync_copy(dst_hbm.at[pl.ds(ep + base, size)], idx)

            @pl.when(c == 0)
            def _():
                pltpu.sync_copy(ms_hbm.at[pl.ds(base, size)], dat)

            @pl.when(c == 1)
            def _():
                pltpu.sync_copy(mv_hbm.at[pl.ds(base, size)], dat)

            pltpu.sync_copy(dat, acc.at[idx], add=True)

        def chunk_body(j, carry):
            do_chunk(base_t + j * CHUNK, CHUNK, idx_v, buf)
            return carry

        lax.fori_loop(0, n_full, chunk_body, 0)
        if tail:
            do_chunk(base_t + n_full * CHUNK, tail, idx_t, buf_t)
        plsc.subcore_barrier()

        def wb_body(q, carry):
            r = row0 + q * wrow
            pltpu.sync_copy(acc.at[pl.ds(r, wrow)], wbuf)

            @pl.when(c == 0)
            def _():
                pltpu.sync_copy(wbuf, outs_hbm.at[pl.ds(r, wrow)])

            @pl.when(c == 1)
            def _():
                pltpu.sync_copy(wbuf, outv_hbm.at[pl.ds(r, wrow)])

            return carry

        lax.fori_loop(0, n_wchunk, wb_body, 0)

    f = pl.kernel(
        body,
        out_type=[pltpu.HBM((n_nodes, 32), jnp.float32),
                  pltpu.HBM((n_nodes, 32), jnp.float32)],
        mesh=mesh,
        scratch_types=[pltpu.VMEM_SHARED((n_nodes, 32), jnp.float32),
                       pltpu.VMEM((CHUNK,), jnp.int32),
                       pltpu.VMEM((CHUNK, 32), jnp.float32),
                       pltpu.VMEM((max(tail, 16),), jnp.int32),
                       pltpu.VMEM((max(tail, 16), 32), jnp.float32),
                       pltpu.VMEM((wrow, 32), jnp.float32),
                       pltpu.SemaphoreType.DMA],
        compiler_params=pltpu.CompilerParams(use_tc_tiling_on_sc=False),
    )
    return f(msg_s, msg_v, dst, zrows)


def _edge_body(be, src_ref, dst_ref, wemb_ref, w0_ref, w1_ref,
               w2_ref, ms_ref, mv_ref):
    a = src_ref[...]
    b = dst_ref[...]
    d = b[:, 0:3] - a[:, 0:3]
    r = jnp.sqrt(jnp.sum(d * d, axis=1, keepdims=True) + 1e-12)
    unit = d / (r + 1e-8)
    sh1 = jnp.sqrt(3.0) * unit
    rs = r + 1e-8
    n = lax.broadcasted_iota(jnp.int32, (1, NB), 1).astype(jnp.float32) + 1.0
    bes = jnp.sqrt(2.0 / RC) * jnp.sin(n * (jnp.pi / RC) * rs) / rs
    u = jnp.clip(r / RC, 0.0, 1.0)
    env = 1.0 - 28.0 * u**6 + 48.0 * u**7 - 21.0 * u**8
    env = jnp.where(u < 1.0, env, 0.0)
    ea = bes * env
    h = jnp.tanh(jnp.dot(ea, w0_ref[...], preferred_element_type=jnp.float32))
    h = jnp.tanh(jnp.dot(h, w1_ref[...], preferred_element_type=jnp.float32))
    w = jnp.dot(h, w2_ref[...], preferred_element_type=jnp.float32)
    spec = a[:, 3:4]
    oh = (spec == lax.broadcasted_iota(jnp.int32, (1, N_ELEM), 1).astype(
        jnp.float32))
    s_src = jnp.dot(oh.astype(jnp.float32), wemb_ref[...],
                    preferred_element_type=jnp.float32)
    ms_ref[...] = w[:, 0:F] * s_src * INV_SQRT_DEG
    mv8 = w[:, F:F + FV] * s_src[:, 0:FV] * INV_SQRT_DEG
    zero8 = jnp.zeros((be, FV), jnp.float32)
    mv_ref[...] = jnp.concatenate(
        [mv8 * sh1[:, 0:1], mv8 * sh1[:, 1:2], mv8 * sh1[:, 2:3], zero8],
        axis=1)


def _tc_edge(srcrow, dstrow, w_emb, w_r0, w_r1, w_r2):
    ep = srcrow.shape[0]
    be = 4000
    grid = ep // be
    full = lambda shape: pl.BlockSpec(shape, lambda i: (0, 0))
    return pl.pallas_call(
        functools.partial(_edge_body, be),
        grid=(grid,),
        in_specs=[pl.BlockSpec((be, 8), lambda i: (i, 0)),
                  pl.BlockSpec((be, 8), lambda i: (i, 0)),
                  full(w_emb.shape), full(w_r0.shape), full(w_r1.shape),
                  full(w_r2.shape)],
        out_specs=[pl.BlockSpec((be, 32), lambda i: (i, 0)),
                   pl.BlockSpec((be, 32), lambda i: (i, 0))],
        out_shape=[jax.ShapeDtypeStruct((ep, 32), jnp.float32),
                   jax.ShapeDtypeStruct((ep, 32), jnp.float32)],
    )(srcrow, dstrow, w_emb, w_r0, w_r1, w_r2)


def _norm_act(x):
    n = jnp.abs(x)
    return x * jnp.tanh(n) / (n + 1e-8)


def _node_body(sp_ref, as_ref, av_ref, wattr, wemb, wsc, wouts, wv2s, wfin,
               wrnl1, wm0, wm1, wm2, out_ref):
    sp = sp_ref[...]
    oh = (sp == lax.broadcasted_iota(jnp.int32, (1, N_ELEM), 1)).astype(
        jnp.float32)
    dot = lambda x, w: jnp.dot(x, w[...], preferred_element_type=jnp.float32)
    s0 = dot(oh, wemb)
    na = dot(oh, wattr)
    sc = dot(s0 * na, wsc)
    av = av_ref[...]
    vnorm = jnp.sqrt(av[:, 0:8]**2 + av[:, 8:16]**2 + av[:, 16:24]**2 + 1e-8)
    s_out = sc + dot(as_ref[...], wouts) + dot(vnorm, wv2s)
    s_out = _norm_act(s_out)
    x = _norm_act(dot(s_out, wfin))
    x = x + _norm_act(dot(x, wrnl1))
    hm = jnp.tanh(dot(x, wm0) * (1.0 / jnp.sqrt(32.0)))
    hm = jnp.tanh(dot(hm, wm1) * (1.0 / jnp.sqrt(64.0)))
    out_ref[...] = dot(hm, wm2) * (1.0 / jnp.sqrt(32.0))


def _tc_node(species2d, agg_s, agg_v, w_attr, w_emb, w_sc, w_out_s, w_v2s,
             w_fin, w_rnl1, w_m0, w_m1, w_m2):
    n = species2d.shape[0]
    bn = 2000
    grid = n // bn
    full = lambda arr: pl.BlockSpec(arr.shape, lambda i: (0, 0))
    return pl.pallas_call(
        _node_body,
        grid=(grid,),
        in_specs=[pl.BlockSpec((bn, 1), lambda i: (i, 0)),
                  pl.BlockSpec((bn, 32), lambda i: (i, 0)),
                  pl.BlockSpec((bn, 32), lambda i: (i, 0)),
                  full(w_attr), full(w_emb), full(w_sc), full(w_out_s),
                  full(w_v2s), full(w_fin), full(w_rnl1), full(w_m0),
                  full(w_m1), full(w_m2)],
        out_specs=pl.BlockSpec((bn, 1), lambda i: (i, 0)),
        out_shape=jax.ShapeDtypeStruct((n, 1), jnp.float32),
    )(species2d, agg_s, agg_v, w_attr, w_emb, w_sc, w_out_s, w_v2s, w_fin,
      w_rnl1, w_m0, w_m1, w_m2)


def kernel(species, coords, edge_index, contributions, W_attr, W_emb, W_r0,
           W_r1, W_r2, W_sc, W_out_s, W_v2s, W_fin, W_rnl1, W_m0, W_m1,
           W_m2):
    n = coords.shape[0]
    tab = jnp.concatenate(
        [coords.astype(jnp.float32),
         species.astype(jnp.float32)[:, None],
         jnp.zeros((n, 4), jnp.float32)], axis=1)
    ei = edge_index.astype(jnp.int32).reshape(-1)
    srcrow, dstrow = _sc_gather(tab, ei)
    ms, mv = _tc_edge(srcrow, dstrow, W_emb, W_r0, W_r1, W_r2)
    zrows = jnp.zeros((125, 32), jnp.float32)
    agg_s, agg_v = _sc_scatter(ms, mv, ei, zrows, n)
    out = _tc_node(species[:, None].astype(jnp.int32), agg_s, agg_v, W_attr,
                   W_emb, W_sc, W_out_s, W_v2s, W_fin, W_rnl1, W_m0, W_m1,
                   W_m2)
    return out


# speedup vs baseline: 32.3656x; 1.6980x over previous
import functools

import jax
import jax.numpy as jnp
from jax import lax
from jax.experimental import pallas as pl
from jax.experimental.pallas import tpu as pltpu
from jax.experimental.pallas import tpu_sc as plsc

NC = 2
NS = 16
NW = NC * NS
CHUNK = 128

N_ELEM = 4
NB = 8
RC = 5.0
F = 32
FV = 8
INV_SQRT_DEG = 0.25


def _sc_gather(tab, edge_index):
    ep = edge_index.shape[0] // 2
    per_w = ep // NW
    n_full = per_w // CHUNK
    has_tail = (per_w % CHUNK) != 0
    mesh = plsc.VectorSubcoreMesh(
        core_axis_name="c", subcore_axis_name="s", num_cores=NC,
        num_subcores=NS)

    def body(tab_hbm, ei_hbm, osrc_hbm, odst_hbm, idx_v, rows_v, sem):
        wid = lax.axis_index("s") * NC + lax.axis_index("c")
        base_w = wid * per_w

        def do_chunk(base):
            pltpu.sync_copy(ei_hbm.at[pl.ds(base, CHUNK)], idx_v)
            pltpu.async_copy(tab_hbm.at[idx_v], rows_v, sem).wait()
            pltpu.sync_copy(rows_v, osrc_hbm.at[pl.ds(base, CHUNK)])
            pltpu.sync_copy(ei_hbm.at[pl.ds(ep + base, CHUNK)], idx_v)
            pltpu.async_copy(tab_hbm.at[idx_v], rows_v, sem).wait()
            pltpu.sync_copy(rows_v, odst_hbm.at[pl.ds(base, CHUNK)])

        def chunk_body(j, carry):
            do_chunk(base_w + j * CHUNK)
            return carry

        lax.fori_loop(0, n_full, chunk_body, 0)
        if has_tail:
            do_chunk(base_w + per_w - CHUNK)

    f = pl.kernel(
        body,
        out_type=[pltpu.HBM((ep, 8), jnp.float32),
                  pltpu.HBM((ep, 8), jnp.float32)],
        mesh=mesh,
        scratch_types=[pltpu.VMEM((CHUNK,), jnp.int32),
                       pltpu.VMEM((CHUNK, 8), jnp.float32),
                       pltpu.SemaphoreType.DMA],
        compiler_params=pltpu.CompilerParams(use_tc_tiling_on_sc=False),
    )
    return f(tab, edge_index)


def _sc_scatter(msg_s, msg_v, dst, zrows, n_nodes):
    ep = dst.shape[0] // 2
    per_t = ep // NS
    n_full = per_t // CHUNK
    tail = per_t % CHUNK
    rows_t = n_nodes // NS
    wrow = 125
    n_wchunk = rows_t // wrow
    mesh = plsc.VectorSubcoreMesh(
        core_axis_name="c", subcore_axis_name="s", num_cores=NC,
        num_subcores=NS)

    def body(ms_hbm, mv_hbm, dst_hbm, z_hbm, outs_hbm, outv_hbm,
             acc, idx_v, buf, idx_t, buf_t, wbuf, sem):
        c = lax.axis_index("c")
        s = lax.axis_index("s")
        base_t = s * per_t
        row0 = s * rows_t

        pltpu.sync_copy(z_hbm, wbuf)

        def zero_body(q, carry):
            pltpu.sync_copy(wbuf, acc.at[pl.ds(row0 + q * wrow, wrow)])
            return carry

        lax.fori_loop(0, n_wchunk, zero_body, 0)
        plsc.subcore_barrier()

        def do_chunk(base, size, idx, dat):
            pltpu.sync_copy(dst_hbm.at[pl.ds(ep + base, size)], idx)

            @pl.when(c == 0)
            def _():
                pltpu.sync_copy(ms_hbm.at[pl.ds(base, size)], dat)

            @pl.when(c == 1)
            def _():
                pltpu.sync_copy(mv_hbm.at[pl.ds(base, size)], dat)

            pltpu.sync_copy(dat, acc.at[idx], add=True)

        def chunk_body(j, carry):
            do_chunk(base_t + j * CHUNK, CHUNK, idx_v, buf)
            return carry

        lax.fori_loop(0, n_full, chunk_body, 0)
        if tail:
            do_chunk(base_t + n_full * CHUNK, tail, idx_t, buf_t)
        plsc.subcore_barrier()

        def wb_body(q, carry):
            r = row0 + q * wrow
            pltpu.sync_copy(acc.at[pl.ds(r, wrow)], wbuf)

            @pl.when(c == 0)
            def _():
                pltpu.sync_copy(wbuf, outs_hbm.at[pl.ds(r, wrow)])

            @pl.when(c == 1)
            def _():
                pltpu.sync_copy(wbuf, outv_hbm.at[pl.ds(r, wrow)])

            return carry

        lax.fori_loop(0, n_wchunk, wb_body, 0)

    f = pl.kernel(
        body,
        out_type=[pltpu.HBM((n_nodes, 32), jnp.float32),
                  pltpu.HBM((n_nodes, 32), jnp.float32)],
        mesh=mesh,
        scratch_types=[pltpu.VMEM_SHARED((n_nodes, 32), jnp.float32),
                       pltpu.VMEM((CHUNK,), jnp.int32),
                       pltpu.VMEM((CHUNK, 32), jnp.float32),
                       pltpu.VMEM((max(tail, 16),), jnp.int32),
                       pltpu.VMEM((max(tail, 16), 32), jnp.float32),
                       pltpu.VMEM((wrow, 32), jnp.float32),
                       pltpu.SemaphoreType.DMA],
        compiler_params=pltpu.CompilerParams(use_tc_tiling_on_sc=False),
    )
    return f(msg_s, msg_v, dst, zrows)


def _edge_body(be, src_ref, dst_ref, wembT_ref, w0T_ref, w1T_ref,
               w2T_ref, ms_ref, mv_ref):
    at = jnp.transpose(src_ref[...][:, 0:4])
    bt = jnp.transpose(dst_ref[...][:, 0:4])
    d = bt[0:3, :] - at[0:3, :]
    r = jnp.sqrt(d[0:1, :] ** 2 + d[1:2, :] ** 2 + d[2:3, :] ** 2 + 1e-12)
    sh1 = d * (jnp.sqrt(3.0) / (r + 1e-8))
    rs = r + 1e-8
    n = lax.broadcasted_iota(jnp.int32, (NB, 1), 0).astype(jnp.float32) + 1.0
    bes = jnp.sqrt(2.0 / RC) * jnp.sin(n * (jnp.pi / RC) * rs) / rs
    u = jnp.clip(r * (1.0 / RC), 0.0, 1.0)
    env = 1.0 - 28.0 * u**6 + 48.0 * u**7 - 21.0 * u**8
    env = jnp.where(u < 1.0, env, 0.0)
    ea = bes * env
    dot = lambda w, x: jnp.dot(w[...], x, preferred_element_type=jnp.float32)
    h = jnp.tanh(dot(w0T_ref, ea))
    h = jnp.tanh(dot(w1T_ref, h))
    w = dot(w2T_ref, h)
    spec = at[3:4, :]
    oh = (spec == lax.broadcasted_iota(jnp.int32, (N_ELEM, 1), 0).astype(
        jnp.float32)).astype(jnp.float32)
    s_src = dot(wembT_ref, oh)
    ms_t = w[0:F, :] * s_src * INV_SQRT_DEG
    mv8 = w[F:F + FV, :] * s_src[0:FV, :] * INV_SQRT_DEG
    mv_t = jnp.concatenate(
        [mv8 * sh1[0:1, :], mv8 * sh1[1:2, :], mv8 * sh1[2:3, :],
         jnp.zeros((FV, be), jnp.float32)], axis=0)
    ms_ref[...] = jnp.transpose(ms_t)
    mv_ref[...] = jnp.transpose(mv_t)


def _tc_edge(srcrow, dstrow, w_emb, w_r0, w_r1, w_r2):
    ep = srcrow.shape[0]
    be = 6400
    grid = ep // be
    full = lambda shape: pl.BlockSpec(shape, lambda i: (0, 0))
    wembT = jnp.transpose(w_emb)
    w0T = jnp.transpose(w_r0)
    w1T = jnp.transpose(w_r1)
    w2T = jnp.transpose(w_r2)
    return pl.pallas_call(
        functools.partial(_edge_body, be),
        grid=(grid,),
        in_specs=[pl.BlockSpec((be, 8), lambda i: (i, 0)),
                  pl.BlockSpec((be, 8), lambda i: (i, 0)),
                  full(wembT.shape), full(w0T.shape), full(w1T.shape),
                  full(w2T.shape)],
        out_specs=[pl.BlockSpec((be, 32), lambda i: (i, 0)),
                   pl.BlockSpec((be, 32), lambda i: (i, 0))],
        out_shape=[jax.ShapeDtypeStruct((ep, 32), jnp.float32),
                   jax.ShapeDtypeStruct((ep, 32), jnp.float32)],
    )(srcrow, dstrow, wembT, w0T, w1T, w2T)


def _norm_act(x):
    n = jnp.abs(x)
    return x * jnp.tanh(n) / (n + 1e-8)


def _node_body(sp_ref, as_ref, av_ref, wattr, wemb, wsc, wouts, wv2s, wfin,
               wrnl1, wm0, wm1, wm2, out_ref):
    sp = sp_ref[...]
    oh = (sp == lax.broadcasted_iota(jnp.int32, (1, N_ELEM), 1)).astype(
        jnp.float32)
    dot = lambda x, w: jnp.dot(x, w[...], preferred_element_type=jnp.float32)
    s0 = dot(oh, wemb)
    na = dot(oh, wattr)
    sc = dot(s0 * na, wsc)
    av = av_ref[...]
    vnorm = jnp.sqrt(av[:, 0:8]**2 + av[:, 8:16]**2 + av[:, 16:24]**2 + 1e-8)
    s_out = sc + dot(as_ref[...], wouts) + dot(vnorm, wv2s)
    s_out = _norm_act(s_out)
    x = _norm_act(dot(s_out, wfin))
    x = x + _norm_act(dot(x, wrnl1))
    hm = jnp.tanh(dot(x, wm0) * (1.0 / jnp.sqrt(32.0)))
    hm = jnp.tanh(dot(hm, wm1) * (1.0 / jnp.sqrt(64.0)))
    out_ref[...] = dot(hm, wm2) * (1.0 / jnp.sqrt(32.0))


def _tc_node(species2d, agg_s, agg_v, w_attr, w_emb, w_sc, w_out_s, w_v2s,
             w_fin, w_rnl1, w_m0, w_m1, w_m2):
    n = species2d.shape[0]
    bn = 2000
    grid = n // bn
    full = lambda arr: pl.BlockSpec(arr.shape, lambda i: (0, 0))
    return pl.pallas_call(
        _node_body,
        grid=(grid,),
        in_specs=[pl.BlockSpec((bn, 1), lambda i: (i, 0)),
                  pl.BlockSpec((bn, 32), lambda i: (i, 0)),
                  pl.BlockSpec((bn, 32), lambda i: (i, 0)),
                  full(w_attr), full(w_emb), full(w_sc), full(w_out_s),
                  full(w_v2s), full(w_fin), full(w_rnl1), full(w_m0),
                  full(w_m1), full(w_m2)],
        out_specs=pl.BlockSpec((bn, 1), lambda i: (i, 0)),
        out_shape=jax.ShapeDtypeStruct((n, 1), jnp.float32),
    )(species2d, agg_s, agg_v, w_attr, w_emb, w_sc, w_out_s, w_v2s, w_fin,
      w_rnl1, w_m0, w_m1, w_m2)


def kernel(species, coords, edge_index, contributions, W_attr, W_emb, W_r0,
           W_r1, W_r2, W_sc, W_out_s, W_v2s, W_fin, W_rnl1, W_m0, W_m1,
           W_m2):
    n = coords.shape[0]
    tab = jnp.concatenate(
        [coords.astype(jnp.float32),
         species.astype(jnp.float32)[:, None],
         jnp.zeros((n, 4), jnp.float32)], axis=1)
    ei = edge_index.astype(jnp.int32).reshape(-1)
    srcrow, dstrow = _sc_gather(tab, ei)
    ms, mv = _tc_edge(srcrow, dstrow, W_emb, W_r0, W_r1, W_r2)
    zrows = jnp.zeros((125, 32), jnp.float32)
    agg_s, agg_v = _sc_scatter(ms, mv, ei, zrows, n)
    out = _tc_node(species[:, None].astype(jnp.int32), agg_s, agg_v, W_attr,
                   W_emb, W_sc, W_out_s, W_v2s, W_fin, W_rnl1, W_m0, W_m1,
                   W_m2)
    return out


# speedup vs baseline: 37.5748x; 1.1609x over previous
import functools

import jax
import jax.numpy as jnp
from jax import lax
from jax.experimental import pallas as pl
from jax.experimental.pallas import tpu as pltpu
from jax.experimental.pallas import tpu_sc as plsc

NC = 2
NS = 16
NW = NC * NS
CHUNK = 128

N_ELEM = 4
NB = 8
RC = 5.0
F = 32
FV = 8
INV_SQRT_DEG = 0.25


SUPER_G = 2048
KG = SUPER_G // CHUNK


def _sc_gather(tab, edge_index):
    ep = edge_index.shape[0] // 2
    per_w = ep // NW
    n_full = per_w // SUPER_G
    has_tail = (per_w % SUPER_G) != 0
    mesh = plsc.VectorSubcoreMesh(
        core_axis_name="c", subcore_axis_name="s", num_cores=NC,
        num_subcores=NS)

    def body(tab_hbm, ei_hbm, osrc_hbm, odst_hbm, idx_v, rows_v, sem):
        wid = lax.axis_index("s") * NC + lax.axis_index("c")
        base_w = wid * per_w

        def do_super(base, off, out_hbm):
            pltpu.sync_copy(ei_hbm.at[pl.ds(off + base, SUPER_G)], idx_v)
            cps = [
                pltpu.async_copy(
                    tab_hbm.at[idx_v.at[pl.ds(k * CHUNK, CHUNK)]],
                    rows_v.at[pl.ds(k * CHUNK, CHUNK)], sem)
                for k in range(KG)]
            for cp in cps:
                cp.wait()
            pltpu.sync_copy(rows_v, out_hbm.at[pl.ds(base, SUPER_G)])

        def chunk_body(j, carry):
            do_super(base_w + j * SUPER_G, 0, osrc_hbm)
            do_super(base_w + j * SUPER_G, ep, odst_hbm)
            return carry

        lax.fori_loop(0, n_full, chunk_body, 0)
        if has_tail:
            do_super(base_w + per_w - SUPER_G, 0, osrc_hbm)
            do_super(base_w + per_w - SUPER_G, ep, odst_hbm)

    f = pl.kernel(
        body,
        out_type=[pltpu.HBM((ep, 8), jnp.float32),
                  pltpu.HBM((ep, 8), jnp.float32)],
        mesh=mesh,
        scratch_types=[pltpu.VMEM((SUPER_G,), jnp.int32),
                       pltpu.VMEM((SUPER_G, 8), jnp.float32),
                       pltpu.SemaphoreType.DMA],
        compiler_params=pltpu.CompilerParams(use_tc_tiling_on_sc=False),
    )
    return f(tab, edge_index)


def _sc_scatter(msg_s, msg_v, dst, zrows, n_nodes):
    ep = dst.shape[0] // 2
    per_t = ep // NS
    n_full = per_t // CHUNK
    tail = per_t % CHUNK
    rows_t = n_nodes // NS
    wrow = 125
    n_wchunk = rows_t // wrow
    mesh = plsc.VectorSubcoreMesh(
        core_axis_name="c", subcore_axis_name="s", num_cores=NC,
        num_subcores=NS)

    def body(ms_hbm, mv_hbm, dst_hbm, z_hbm, outs_hbm, outv_hbm,
             acc, idx_v, buf, idx_t, buf_t, wbuf, sem):
        c = lax.axis_index("c")
        s = lax.axis_index("s")
        base_t = s * per_t
        row0 = s * rows_t

        pltpu.sync_copy(z_hbm, wbuf)

        def zero_body(q, carry):
            pltpu.sync_copy(wbuf, acc.at[pl.ds(row0 + q * wrow, wrow)])
            return carry

        lax.fori_loop(0, n_wchunk, zero_body, 0)
        plsc.subcore_barrier()

        def do_chunk(base, size, idx, dat):
            pltpu.sync_copy(dst_hbm.at[pl.ds(ep + base, size)], idx)

            @pl.when(c == 0)
            def _():
                pltpu.sync_copy(ms_hbm.at[pl.ds(base, size)], dat)

            @pl.when(c == 1)
            def _():
                pltpu.sync_copy(mv_hbm.at[pl.ds(base, size)], dat)

            pltpu.sync_copy(dat, acc.at[idx], add=True)

        def chunk_body(j, carry):
            do_chunk(base_t + j * CHUNK, CHUNK, idx_v, buf)
            return carry

        lax.fori_loop(0, n_full, chunk_body, 0)
        if tail:
            do_chunk(base_t + n_full * CHUNK, tail, idx_t, buf_t)
        plsc.subcore_barrier()

        def wb_body(q, carry):
            r = row0 + q * wrow
            pltpu.sync_copy(acc.at[pl.ds(r, wrow)], wbuf)

            @pl.when(c == 0)
            def _():
                pltpu.sync_copy(wbuf, outs_hbm.at[pl.ds(r, wrow)])

            @pl.when(c == 1)
            def _():
                pltpu.sync_copy(wbuf, outv_hbm.at[pl.ds(r, wrow)])

            return carry

        lax.fori_loop(0, n_wchunk, wb_body, 0)

    f = pl.kernel(
        body,
        out_type=[pltpu.HBM((n_nodes, 32), jnp.float32),
                  pltpu.HBM((n_nodes, 32), jnp.float32)],
        mesh=mesh,
        scratch_types=[pltpu.VMEM_SHARED((n_nodes, 32), jnp.float32),
                       pltpu.VMEM((CHUNK,), jnp.int32),
                       pltpu.VMEM((CHUNK, 32), jnp.float32),
                       pltpu.VMEM((max(tail, 16),), jnp.int32),
                       pltpu.VMEM((max(tail, 16), 32), jnp.float32),
                       pltpu.VMEM((wrow, 32), jnp.float32),
                       pltpu.SemaphoreType.DMA],
        compiler_params=pltpu.CompilerParams(use_tc_tiling_on_sc=False),
    )
    return f(msg_s, msg_v, dst, zrows)


def _edge_body(be, src_ref, dst_ref, wembT_ref, w0T_ref, w1T_ref,
               w2T_ref, ms_ref, mv_ref):
    at = jnp.transpose(src_ref[...][:, 0:4])
    bt = jnp.transpose(dst_ref[...][:, 0:4])
    d = bt[0:3, :] - at[0:3, :]
    r = jnp.sqrt(d[0:1, :] ** 2 + d[1:2, :] ** 2 + d[2:3, :] ** 2 + 1e-12)
    sh1 = d * (jnp.sqrt(3.0) / (r + 1e-8))
    rs = r + 1e-8
    n = lax.broadcasted_iota(jnp.int32, (NB, 1), 0).astype(jnp.float32) + 1.0
    bes = jnp.sqrt(2.0 / RC) * jnp.sin(n * (jnp.pi / RC) * rs) / rs
    u = jnp.clip(r * (1.0 / RC), 0.0, 1.0)
    env = 1.0 - 28.0 * u**6 + 48.0 * u**7 - 21.0 * u**8
    env = jnp.where(u < 1.0, env, 0.0)
    ea = bes * env
    dot = lambda w, x: jnp.dot(w[...], x, preferred_element_type=jnp.float32)
    h = jnp.tanh(dot(w0T_ref, ea))
    h = jnp.tanh(dot(w1T_ref, h))
    w = dot(w2T_ref, h)
    spec = at[3:4, :]
    oh = (spec == lax.broadcasted_iota(jnp.int32, (N_ELEM, 1), 0).astype(
        jnp.float32)).astype(jnp.float32)
    s_src = dot(wembT_ref, oh)
    ms_t = w[0:F, :] * s_src * INV_SQRT_DEG
    mv8 = w[F:F + FV, :] * s_src[0:FV, :] * INV_SQRT_DEG
    mv_t = jnp.concatenate(
        [mv8 * sh1[0:1, :], mv8 * sh1[1:2, :], mv8 * sh1[2:3, :],
         jnp.zeros((FV, be), jnp.float32)], axis=0)
    ms_ref[...] = jnp.transpose(ms_t)
    mv_ref[...] = jnp.transpose(mv_t)


def _tc_edge(srcrow, dstrow, w_emb, w_r0, w_r1, w_r2):
    ep = srcrow.shape[0]
    be = 6400
    grid = ep // be
    full = lambda shape: pl.BlockSpec(shape, lambda i: (0, 0))
    wembT = jnp.transpose(w_emb)
    w0T = jnp.transpose(w_r0)
    w1T = jnp.transpose(w_r1)
    w2T = jnp.transpose(w_r2)
    return pl.pallas_call(
        functools.partial(_edge_body, be),
        grid=(grid,),
        in_specs=[pl.BlockSpec((be, 8), lambda i: (i, 0)),
                  pl.BlockSpec((be, 8), lambda i: (i, 0)),
                  full(wembT.shape), full(w0T.shape), full(w1T.shape),
                  full(w2T.shape)],
        out_specs=[pl.BlockSpec((be, 32), lambda i: (i, 0)),
                   pl.BlockSpec((be, 32), lambda i: (i, 0))],
        out_shape=[jax.ShapeDtypeStruct((ep, 32), jnp.float32),
                   jax.ShapeDtypeStruct((ep, 32), jnp.float32)],
    )(srcrow, dstrow, wembT, w0T, w1T, w2T)


def _norm_act(x):
    n = jnp.abs(x)
    return x * jnp.tanh(n) / (n + 1e-8)


def _node_body(sp_ref, as_ref, av_ref, wattr, wemb, wsc, wouts, wv2s, wfin,
               wrnl1, wm0, wm1, wm2, out_ref):
    sp = sp_ref[...]
    oh = (sp == lax.broadcasted_iota(jnp.int32, (1, N_ELEM), 1)).astype(
        jnp.float32)
    dot = lambda x, w: jnp.dot(x, w[...], preferred_element_type=jnp.float32)
    s0 = dot(oh, wemb)
    na = dot(oh, wattr)
    sc = dot(s0 * na, wsc)
    av = av_ref[...]
    vnorm = jnp.sqrt(av[:, 0:8]**2 + av[:, 8:16]**2 + av[:, 16:24]**2 + 1e-8)
    s_out = sc + dot(as_ref[...], wouts) + dot(vnorm, wv2s)
    s_out = _norm_act(s_out)
    x = _norm_act(dot(s_out, wfin))
    x = x + _norm_act(dot(x, wrnl1))
    hm = jnp.tanh(dot(x, wm0) * (1.0 / jnp.sqrt(32.0)))
    hm = jnp.tanh(dot(hm, wm1) * (1.0 / jnp.sqrt(64.0)))
    out_ref[...] = dot(hm, wm2) * (1.0 / jnp.sqrt(32.0))


def _tc_node(species2d, agg_s, agg_v, w_attr, w_emb, w_sc, w_out_s, w_v2s,
             w_fin, w_rnl1, w_m0, w_m1, w_m2):
    n = species2d.shape[0]
    bn = 2000
    grid = n // bn
    full = lambda arr: pl.BlockSpec(arr.shape, lambda i: (0, 0))
    return pl.pallas_call(
        _node_body,
        grid=(grid,),
        in_specs=[pl.BlockSpec((bn, 1), lambda i: (i, 0)),
                  pl.BlockSpec((bn, 32), lambda i: (i, 0)),
                  pl.BlockSpec((bn, 32), lambda i: (i, 0)),
                  full(w_attr), full(w_emb), full(w_sc), full(w_out_s),
                  full(w_v2s), full(w_fin), full(w_rnl1), full(w_m0),
                  full(w_m1), full(w_m2)],
        out_specs=pl.BlockSpec((bn, 1), lambda i: (i, 0)),
        out_shape=jax.ShapeDtypeStruct((n, 1), jnp.float32),
    )(species2d, agg_s, agg_v, w_attr, w_emb, w_sc, w_out_s, w_v2s, w_fin,
      w_rnl1, w_m0, w_m1, w_m2)


def kernel(species, coords, edge_index, contributions, W_attr, W_emb, W_r0,
           W_r1, W_r2, W_sc, W_out_s, W_v2s, W_fin, W_rnl1, W_m0, W_m1,
           W_m2):
    n = coords.shape[0]
    tab = jnp.concatenate(
        [coords.astype(jnp.float32),
         species.astype(jnp.float32)[:, None],
         jnp.zeros((n, 4), jnp.float32)], axis=1)
    ei = edge_index.astype(jnp.int32).reshape(-1)
    srcrow, dstrow = _sc_gather(tab, ei)
    ms, mv = _tc_edge(srcrow, dstrow, W_emb, W_r0, W_r1, W_r2)
    zrows = jnp.zeros((125, 32), jnp.float32)
    agg_s, agg_v = _sc_scatter(ms, mv, ei, zrows, n)
    out = _tc_node(species[:, None].astype(jnp.int32), agg_s, agg_v, W_attr,
                   W_emb, W_sc, W_out_s, W_v2s, W_fin, W_rnl1, W_m0, W_m1,
                   W_m2)
    return out
